# baseline (speedup 1.0000x reference)
import functools

import jax
import jax.numpy as jnp
from jax import lax
from jax.experimental import pallas as pl
from jax.experimental.pallas import tpu as pltpu
from jax.experimental.pallas import tpu_sc as plsc

N = 10000
E = 160000
HID = 128
N_PAD = 10240
E_PAD = 163840
NC, NS = 2, 16
NW = NC * NS
CH = 128
PER_W = E_PAD // NW
N_CHUNK = PER_W // CH
_f32 = jnp.float32


def _sc_mesh():
    return plsc.VectorSubcoreMesh(core_axis_name="c", subcore_axis_name="s",
                                  num_cores=NC, num_subcores=NS)


def _sc_gather_combine(tab_d, tab_s, idx_d, idx_s, out_w, combine):
    W = tab_d.shape[1]
    dt = tab_d.dtype
    ch = 64
    nch = PER_W // ch
    NB = 4
    out_t = jax.ShapeDtypeStruct((E_PAD, out_w), dt)

    @functools.partial(
        pl.kernel,
        out_type=out_t,
        mesh=_sc_mesh(),
        scratch_types=[
            [pltpu.VMEM((ch,), jnp.int32)] * NB,
            [pltpu.VMEM((ch,), jnp.int32)] * NB,
            [pltpu.VMEM((ch, W), dt)] * NB,
            [pltpu.VMEM((ch, W), dt)] * NB,
            [pltpu.VMEM((ch, out_w), dt)] * NB,
            [pltpu.SemaphoreType.DMA] * NB,
            [pltpu.SemaphoreType.DMA] * NB,
            [pltpu.SemaphoreType.DMA] * NB,
        ],
    )
    def k(td_hbm, ts_hbm, id_hbm, is_hbm, go_hbm,
          iv_d, iv_s, rows_d, rows_s, rows_o, si, sg, so):
        wid = lax.axis_index("s") * NC + lax.axis_index("c")
        w0 = wid * PER_W

        def issue_idx(i, b):
            pltpu.async_copy(id_hbm.at[pl.ds(w0 + i * ch, ch)], iv_d[b], si[b])
            pltpu.async_copy(is_hbm.at[pl.ds(w0 + i * ch, ch)], iv_s[b], si[b])

        def drain_idx(b):
            pltpu.make_async_copy(id_hbm.at[pl.ds(0, ch)], iv_d[b], si[b]).wait()
            pltpu.make_async_copy(is_hbm.at[pl.ds(0, ch)], iv_s[b], si[b]).wait()

        def issue_gather(b):
            pltpu.async_copy(td_hbm.at[iv_d[b]], rows_d[b], sg[b])
            pltpu.async_copy(ts_hbm.at[iv_s[b]], rows_s[b], sg[b])

        def drain_gather(b):
            pltpu.make_async_copy(td_hbm.at[iv_d[b]], rows_d[b], sg[b]).wait()
            pltpu.make_async_copy(ts_hbm.at[iv_s[b]], rows_s[b], sg[b]).wait()

        def issue_wb(j, b):
            pltpu.async_copy(rows_o[b], go_hbm.at[pl.ds(w0 + j * ch, ch)], so[b])

        def drain_wb(b):
            pltpu.make_async_copy(rows_o[b], go_hbm.at[pl.ds(0, ch)], so[b]).wait()

        for q in range(NB):
            issue_idx(q, q)

        def body(s, carry):
            for q in range(NB):
                i = NB * s + q

                @pl.when(s >= 1)
                def _():
                    drain_wb(q)

                drain_idx(q)
                issue_gather(q)
                j = i - 2
                bj = (q + 2) % NB

                @pl.when(j >= 0)
                def _():
                    drain_gather(bj)
                    combine(rows_d[bj], rows_s[bj], rows_o[bj], ch)
                    issue_wb(j, bj)

                    @pl.when(j + NB < nch)
                    def _():
                        issue_idx(j + NB, bj)
            return carry

        lax.fori_loop(0, nch // NB, body, 0)
        for jq in (nch - 2, nch - 1):
            b = jq % NB
            drain_gather(b)
            combine(rows_d[b], rows_s[b], rows_o[b], ch)
            issue_wb(jq, b)
        for b in range(NB):
            drain_wb(b)

    return k(tab_d, tab_s, idx_d, idx_s)


def _combine_sum(rows_d, rows_s, rows_o, ch):
    def row(rr, carry):
        for c in range(HID // 16):
            sl = pl.ds(c * 16, 16)
            rows_o[rr, sl] = rows_d[rr, sl] + rows_s[rr, sl]
        return carry

    lax.fori_loop(0, ch, row, 0)


def _sc_gather_sum(tab_d, tab_s, idx_d, idx_s):
    return _sc_gather_combine(tab_d, tab_s, idx_d, idx_s, HID, _combine_sum)


def _make_geo_combine(ch):
    del ch

    def geo_combine(rows_d, rows_s, rows_o, ch):
        lane = lax.iota(jnp.int32, 16)

        def row(rr, carry):
            sl = pl.ds(0, 16)
            d = rows_d[rr, sl] - rows_s[rr, sl]
            rows_o[rr, sl] = jnp.where(lane == 9, rows_s[rr, sl], d)
            return carry

        lax.fori_loop(0, ch, row, 0)

    return geo_combine


def _sc_gather_geo(tab, idx_d, idx_s):
    return _sc_gather_combine(tab, tab, idx_d, idx_s, 16, _make_geo_combine(64))


def _sc_scatter_add(vals, idx, zeros_hbm):
    W = vals.shape[1]
    dt = vals.dtype
    ch = 64
    nch = PER_W // ch
    NB = 4
    rows_per_sub = N_PAD // NS

    @functools.partial(
        pl.kernel,
        out_type=jax.ShapeDtypeStruct((NC, N_PAD, W), dt),
        mesh=_sc_mesh(),
        scratch_types=[
            [pltpu.VMEM((ch,), jnp.int32)] * NB,
            [pltpu.VMEM((ch, W), dt)] * NB,
            pltpu.VMEM_SHARED((N_PAD, W), dt),
            [pltpu.SemaphoreType.DMA] * NB,
            [pltpu.SemaphoreType.DMA] * NB,
        ],
    )
    def k(v_hbm, i_hbm, z_hbm, out_hbm, iv, rows, acc, si, ss):
        cid = lax.axis_index("c")
        sid = lax.axis_index("s")
        wid = sid * NC + cid
        w0 = wid * PER_W
        sl = pl.ds(sid * rows_per_sub, rows_per_sub)

        def issue_loads(i, b):
            pltpu.async_copy(i_hbm.at[pl.ds(w0 + i * ch, ch)], iv[b], si[b])
            pltpu.async_copy(v_hbm.at[pl.ds(w0 + i * ch, ch)], rows[b], si[b])

        def drain_loads(b):
            pltpu.make_async_copy(i_hbm.at[pl.ds(0, ch)], iv[b], si[b]).wait()
            pltpu.make_async_copy(v_hbm.at[pl.ds(0, ch)], rows[b], si[b]).wait()

        def drain_scatter(b):
            pltpu.make_async_copy(rows[b], acc.at[iv[b]], ss[b]).wait()

        for q in range(NB):
            issue_loads(q, q)
        pltpu.sync_copy(z_hbm.at[sl], acc.at[sl])
        plsc.subcore_barrier()

        def body(s, carry):
            for q in range(NB):
                i = NB * s + q
                drain_loads(q)
                pltpu.async_copy(rows[q], acc.at[iv[q]], ss[q], add=True)
                j = i - 2
                bj = (q + 2) % NB

                @pl.when(j >= 0)
                def _():
                    drain_scatter(bj)

                    @pl.when(j + NB < nch)
                    def _():
                        issue_loads(j + NB, bj)
            return carry

        lax.fori_loop(0, nch // NB, body, 0)
        for jq in (nch - 2, nch - 1):
            drain_scatter(jq % NB)
        plsc.subcore_barrier()
        pltpu.sync_copy(acc.at[sl], out_hbm.at[cid, sl])

    return k(vals, idx, zeros_hbm)


def _sc_count(idx, zeros_hbm, ones_hbm):
    W = ones_hbm.shape[1]
    rows_per_sub = N_PAD // NS

    @functools.partial(
        pl.kernel,
        out_type=jax.ShapeDtypeStruct((NC, N_PAD, W), _f32),
        mesh=_sc_mesh(),
        scratch_types=[
            pltpu.VMEM((CH,), jnp.int32),
            pltpu.VMEM((CH, W), _f32),
            pltpu.VMEM_SHARED((N_PAD, W), _f32),
        ],
    )
    def k(i_hbm, z_hbm, o_hbm, out_hbm, iv, rows, acc):
        cid = lax.axis_index("c")
        sid = lax.axis_index("s")
        wid = sid * NC + cid
        sl = pl.ds(sid * rows_per_sub, rows_per_sub)
        pltpu.sync_copy(z_hbm.at[sl], acc.at[sl])
        pltpu.sync_copy(o_hbm, rows)
        plsc.subcore_barrier()

        def body(i, carry):
            base = wid * PER_W + i * CH
            pltpu.sync_copy(i_hbm.at[pl.ds(base, CH)], iv)
            pltpu.sync_copy(rows, acc.at[iv], add=True)
            return carry

        lax.fori_loop(0, N_CHUNK, body, 0)
        plsc.subcore_barrier()
        pltpu.sync_copy(acc.at[sl], out_hbm.at[cid, sl])

    return k(idx, zeros_hbm, ones_hbm)


def _full(shape):
    return pl.BlockSpec(shape, lambda i: tuple(0 for _ in shape))


def _embed_call(feats, w1, b1, w2, b2, wd, ws):
    BLK = 1024

    def body(f, w1r, b1r, w2r, b2r, wdr, wsr, h_o, pd_o, ps_o):
        x = f[...]
        h1 = jnp.maximum(
            jnp.dot(x, w1r[...], preferred_element_type=_f32) + b1r[...], 0.0)
        h = jnp.maximum(
            jnp.dot(h1, w2r[...], preferred_element_type=_f32) + b2r[...], 0.0)
        h_o[...] = h
        pd_o[...] = jnp.dot(h, wdr[...], preferred_element_type=_f32)
        ps_o[...] = jnp.dot(h, wsr[...], preferred_element_type=_f32)

    return pl.pallas_call(
        body,
        grid=(N_PAD // BLK,),
        in_specs=[
            pl.BlockSpec((BLK, 32), lambda i: (i, 0)),
            _full((32, HID)), _full((1, HID)),
            _full((HID, HID)), _full((1, HID)),
            _full((HID, HID)), _full((HID, HID)),
        ],
        out_specs=[pl.BlockSpec((BLK, HID), lambda i: (i, 0))] * 3,
        out_shape=[jax.ShapeDtypeStruct((N_PAD, HID), _f32)] * 3,
    )(feats, w1, b1, w2, b2, wd, ws)


def _geo_call(graw, dom_row):
    BLK = 2048

    def body(gr, domr, g_o):
        d = gr[...]
        dom = domr[...]
        w = d - dom * jnp.round(d / dom)
        ci = lax.broadcasted_iota(jnp.int32, d.shape, 1)
        g_o[...] = jnp.where(ci < 3, w, d)

    return pl.pallas_call(
        body,
        grid=(E_PAD // BLK,),
        in_specs=[
            pl.BlockSpec((BLK, 16), lambda i: (i, 0)),
            _full((1, 16)),
        ],
        out_specs=pl.BlockSpec((BLK, 16), lambda i: (i, 0)),
        out_shape=jax.ShapeDtypeStruct((E_PAD, 16), _f32),
    )(graw, dom_row)


def _edge_call(gsum, geo, wgeo, b1, w2, b2):
    BLK = 2048

    def body(gr, ger, wgr, b1r, w2r, b2r, m_o):
        h1 = jnp.maximum(
            gr[...]
            + jnp.dot(ger[...], wgr[...], preferred_element_type=_f32)
            + b1r[...], 0.0)
        m_o[...] = jnp.maximum(
            jnp.dot(h1, w2r[...], preferred_element_type=_f32) + b2r[...], 0.0)

    return pl.pallas_call(
        body,
        grid=(E_PAD // BLK,),
        in_specs=[
            pl.BlockSpec((BLK, HID), lambda i: (i, 0)),
            pl.BlockSpec((BLK, 16), lambda i: (i, 0)),
            _full((16, HID)), _full((1, HID)),
            _full((HID, HID)), _full((1, HID)),
        ],
        out_specs=pl.BlockSpec((BLK, HID), lambda i: (i, 0)),
        out_shape=jax.ShapeDtypeStruct((E_PAD, HID), _f32),
    )(gsum, geo, wgeo, b1, w2, b2)


def _node_call(h, aggp, degp, wa, wb, b1, w2, b2, g, be, wd, ws):
    BLK = 1024

    def body(hr, ar, dr, war, wbr, b1r, w2r, b2r, gr, ber, wdr, wsr,
             h_o, pd_o, ps_o):
        h = hr[...]
        deg = jnp.maximum(dr[0, :, 0:1] + dr[1, :, 0:1], 1.0)
        agg = (ar[0].astype(_f32) + ar[1].astype(_f32)) / deg
        u1 = jnp.maximum(
            jnp.dot(h, war[...], preferred_element_type=_f32)
            + jnp.dot(agg, wbr[...], preferred_element_type=_f32)
            + b1r[...], 0.0)
        u = jnp.dot(u1, w2r[...], preferred_element_type=_f32) + b2r[...]
        mu = jnp.mean(u, axis=-1, keepdims=True)
        var = jnp.mean((u - mu) ** 2, axis=-1, keepdims=True)
        un = (u - mu) * lax.rsqrt(var + 1e-5) * gr[...] + ber[...]
        hn = h + un
        h_o[...] = hn
        pd_o[...] = jnp.dot(hn, wdr[...], preferred_element_type=_f32)
        ps_o[...] = jnp.dot(hn, wsr[...], preferred_element_type=_f32)

    return pl.pallas_call(
        body,
        grid=(N_PAD // BLK,),
        in_specs=[
            pl.BlockSpec((BLK, HID), lambda i: (i, 0)),
            pl.BlockSpec((NC, BLK, HID), lambda i: (0, i, 0)),
            pl.BlockSpec((NC, BLK, 16), lambda i: (0, i, 0)),
            _full((HID, HID)), _full((HID, HID)), _full((1, HID)),
            _full((HID, HID)), _full((1, HID)),
            _full((1, HID)), _full((1, HID)),
            _full((HID, HID)), _full((HID, HID)),
        ],
        out_specs=[pl.BlockSpec((BLK, HID), lambda i: (i, 0))] * 3,
        out_shape=[jax.ShapeDtypeStruct((N_PAD, HID), _f32)] * 3,
    )(h, aggp, degp, wa, wb, b1, w2, b2, g, be, wd, ws)


def _head_call(h, nf, ow1, ob1, ow2p, ob2p, scale_row, dn_row):
    BLK = 1024

    def body(hr, nfr, w1r, b1r, w2r, b2r, scr, dnr, pred_o, hsum_o):
        i = pl.program_id(0)
        h = hr[...]
        o1 = jnp.maximum(
            jnp.dot(h, w1r[...], preferred_element_type=_f32) + b1r[...], 0.0)
        o = jnp.dot(o1, w2r[...], preferred_element_type=_f32) + b2r[...]
        base = o * scr[...] + nfr[...]
        dn = dnr[...]
        rem = base - jnp.floor(base / dn) * dn
        ci = lax.broadcasted_iota(jnp.int32, base.shape, 1)
        pred_o[...] = jnp.where(ci < 3, rem, base)
        ri = lax.broadcasted_iota(jnp.int32, (BLK, 1), 0) + i * BLK
        hm = jnp.where(ri < N, h, 0.0)
        part = jnp.sum(hm, axis=0, keepdims=True)

        @pl.when(i == 0)
        def _init():
            hsum_o[...] = jnp.zeros_like(hsum_o)

        hsum_o[...] += part

    return pl.pallas_call(
        body,
        grid=(N_PAD // BLK,),
        in_specs=[
            pl.BlockSpec((BLK, HID), lambda i: (i, 0)),
            pl.BlockSpec((BLK, HID), lambda i: (i, 0)),
            _full((HID, HID)), _full((1, HID)),
            _full((HID, HID)), _full((1, HID)),
            _full((1, HID)), _full((1, HID)),
        ],
        out_specs=[
            pl.BlockSpec((BLK, HID), lambda i: (i, 0)),
            pl.BlockSpec((1, HID), lambda i: (0, 0)),
        ],
        out_shape=[
            jax.ShapeDtypeStruct((N_PAD, HID), _f32),
            jax.ShapeDtypeStruct((1, HID), _f32),
        ],
    )(h, nf, ow1, ob1, ow2p, ob2p, scale_row, dn_row)


def _macro_call(hsum, w1, b1, w2p, b2p):
    def body(hs, w1r, b1r, w2r, b2r, o):
        hm = jnp.broadcast_to(hs[...] * (1.0 / N), (8, HID))
        z1 = jnp.maximum(
            jnp.dot(hm, w1r[...], preferred_element_type=_f32) + b1r[...], 0.0)
        o[...] = jnp.dot(z1, w2r[...], preferred_element_type=_f32) + b2r[...]

    return pl.pallas_call(
        body,
        out_shape=jax.ShapeDtypeStruct((8, HID), _f32),
    )(hsum, w1, b1, w2p, b2p)


def kernel(pos, v, r, t, x_global, domain, domain_next, t_next,
           edge_index, batch, params):
    gfeat = jnp.concatenate([domain, t, x_global, domain_next, t_next])

    src = edge_index[0]
    dst = edge_index[1]
    pad_e = E_PAD - E
    src_p = jnp.concatenate([src, jnp.zeros((pad_e,), jnp.int32)])
    dst_p = jnp.concatenate([dst, jnp.full((pad_e,), N, jnp.int32)])

    T = jnp.concatenate([pos, v, r], axis=1)
    T = jnp.pad(T, ((0, N_PAD - N), (0, HID - 10)))

    feats = jnp.concatenate([r, v, jnp.broadcast_to(gfeat, (N, 12))], axis=1)
    feats = jnp.pad(feats, ((0, N_PAD - N), (0, 32 - 19)))

    Wd_l, Ws_l, Wgeo_l, pb1_l, pW2_l, pb2_l = [], [], [], [], [], []
    A_l, B_l, sb1_l, sW2_l, sb2_l, g_l, be_l = [], [], [], [], [], [], []
    for lp in params["layers"]:
        W1 = lp["phi_W1"]
        Wd_l.append(W1[0:HID])
        Ws_l.append(W1[HID:2 * HID])
        Wgeo_l.append(jnp.pad(W1[2 * HID:2 * HID + 10], ((0, 6), (0, 0))))
        pb1_l.append((lp["phi_b1"] + gfeat @ W1[2 * HID + 10:])[None])
        pW2_l.append(lp["phi_W2"])
        pb2_l.append(lp["phi_b2"][None])
        A_l.append(lp["psi_W1"][0:HID])
        B_l.append(lp["psi_W1"][HID:2 * HID])
        sb1_l.append(lp["psi_b1"][None])
        sW2_l.append(lp["psi_W2"])
        sb2_l.append(lp["psi_b2"][None])
        g_l.append(lp["ln_g"][None])
        be_l.append(lp["ln_b"][None])

    emb_W1p = jnp.pad(params["emb_W1"], ((0, 32 - 19), (0, 0)))

    zeros_np32 = jnp.zeros((N_PAD, HID), _f32)
    zeros_n16 = jnp.zeros((N_PAD, 16), _f32)
    ones_ch = jnp.ones((CH, 16), _f32)
    dom_row = jnp.concatenate([domain, jnp.ones((13,), _f32)])[None]

    geo_raw = _sc_gather_geo(T, dst_p, src_p)
    geo = _geo_call(geo_raw, dom_row)
    degp = _sc_count(dst_p, zeros_n16, ones_ch)
    h, Pd, Ps = _embed_call(feats, emb_W1p, params["emb_b1"][None],
                            params["emb_W2"], params["emb_b2"][None],
                            Wd_l[0], Ws_l[0])
    for l in range(6):
        Gsum = _sc_gather_sum(Pd, Ps, dst_p, src_p)
        m = _edge_call(Gsum, geo, Wgeo_l[l], pb1_l[l], pW2_l[l], pb2_l[l])
        aggp = _sc_scatter_add(m, dst_p, zeros_np32)
        nl = (l + 1) % 6
        h, Pd, Ps = _node_call(h, aggp, degp, A_l[l], B_l[l], sb1_l[l],
                               sW2_l[l], sb2_l[l], g_l[l], be_l[l],
                               Wd_l[nl], Ws_l[nl])

    nf = jnp.pad(jnp.concatenate([pos, v], axis=1),
                 ((0, N_PAD - N), (0, HID - 9)))
    scale_row = jnp.concatenate([
        jnp.full((6,), 0.001, _f32), jnp.full((3,), 0.01, _f32),
        jnp.zeros((HID - 9,), _f32)])[None]
    dn_row = jnp.concatenate([domain_next, jnp.ones((HID - 3,), _f32)])[None]
    ow2p = jnp.pad(params["out_W2"], ((0, 0), (0, HID - 9)))
    ob2p = jnp.pad(params["out_b2"], (0, HID - 9))[None]
    pred, hsum = _head_call(h, nf, params["out_W1"], params["out_b1"][None],
                            ow2p, ob2p, scale_row, dn_row)
    mw2p = jnp.pad(params["mac_W2"], ((0, 0), (0, HID - 3)))
    mb2p = jnp.pad(params["mac_b2"], (0, HID - 3))[None]
    macro8 = _macro_call(hsum, params["mac_W1"], params["mac_b1"][None],
                         mw2p, mb2p)

    return (pred[:N, 0:3], pred[:N, 3:9], macro8[0, 0:3])


# speedup vs baseline: 1.0200x; 1.0200x over previous
import functools

import jax
import jax.numpy as jnp
from jax import lax
from jax.experimental import pallas as pl
from jax.experimental.pallas import tpu as pltpu
from jax.experimental.pallas import tpu_sc as plsc

N = 10000
E = 160000
HID = 128
N_PAD = 10240
E_PAD = 163840
NC, NS = 2, 16
NW = NC * NS
CH = 128
PER_W = E_PAD // NW
N_CHUNK = PER_W // CH
_f32 = jnp.float32


def _sc_mesh():
    return plsc.VectorSubcoreMesh(core_axis_name="c", subcore_axis_name="s",
                                  num_cores=NC, num_subcores=NS)


def _sc_gather_combine(tab_d, tab_s, idx_d, idx_s, out_w, combine):
    W = tab_d.shape[1]
    dt = tab_d.dtype
    ch = 64
    nch = PER_W // ch
    NB = 4
    out_t = jax.ShapeDtypeStruct((E_PAD, out_w), dt)

    @functools.partial(
        pl.kernel,
        out_type=out_t,
        mesh=_sc_mesh(),
        scratch_types=[
            [pltpu.VMEM((ch,), jnp.int32)] * NB,
            [pltpu.VMEM((ch,), jnp.int32)] * NB,
            [pltpu.VMEM((ch, W), dt)] * NB,
            [pltpu.VMEM((ch, W), dt)] * NB,
            [pltpu.VMEM((ch, out_w), dt)] * NB,
            [pltpu.SemaphoreType.DMA] * NB,
            [pltpu.SemaphoreType.DMA] * NB,
            [pltpu.SemaphoreType.DMA] * NB,
        ],
    )
    def k(td_hbm, ts_hbm, id_hbm, is_hbm, go_hbm,
          iv_d, iv_s, rows_d, rows_s, rows_o, si, sg, so):
        wid = lax.axis_index("s") * NC + lax.axis_index("c")
        w0 = wid * PER_W

        def issue_idx(i, b):
            pltpu.async_copy(id_hbm.at[pl.ds(w0 + i * ch, ch)], iv_d[b], si[b])
            pltpu.async_copy(is_hbm.at[pl.ds(w0 + i * ch, ch)], iv_s[b], si[b])

        def drain_idx(b):
            pltpu.make_async_copy(id_hbm.at[pl.ds(0, ch)], iv_d[b], si[b]).wait()
            pltpu.make_async_copy(is_hbm.at[pl.ds(0, ch)], iv_s[b], si[b]).wait()

        def issue_gather(b):
            pltpu.async_copy(td_hbm.at[iv_d[b]], rows_d[b], sg[b])
            pltpu.async_copy(ts_hbm.at[iv_s[b]], rows_s[b], sg[b])

        def drain_gather(b):
            pltpu.make_async_copy(td_hbm.at[iv_d[b]], rows_d[b], sg[b]).wait()
            pltpu.make_async_copy(ts_hbm.at[iv_s[b]], rows_s[b], sg[b]).wait()

        def issue_wb(j, b):
            pltpu.async_copy(rows_o[b], go_hbm.at[pl.ds(w0 + j * ch, ch)], so[b])

        def drain_wb(b):
            pltpu.make_async_copy(rows_o[b], go_hbm.at[pl.ds(0, ch)], so[b]).wait()

        for q in range(NB):
            issue_idx(q, q)

        def body(s, carry):
            for q in range(NB):
                i = NB * s + q

                @pl.when(s >= 1)
                def _():
                    drain_wb(q)

                drain_idx(q)
                issue_gather(q)
                j = i - 2
                bj = (q + 2) % NB

                @pl.when(j >= 0)
                def _():
                    drain_gather(bj)
                    combine(rows_d[bj], rows_s[bj], rows_o[bj], ch)
                    issue_wb(j, bj)

                    @pl.when(j + NB < nch)
                    def _():
                        issue_idx(j + NB, bj)
            return carry

        lax.fori_loop(0, nch // NB, body, 0)
        for jq in (nch - 2, nch - 1):
            b = jq % NB
            drain_gather(b)
            combine(rows_d[b], rows_s[b], rows_o[b], ch)
            issue_wb(jq, b)
        for b in range(NB):
            drain_wb(b)

    return k(tab_d, tab_s, idx_d, idx_s)


def _combine_sum(rows_d, rows_s, rows_o, ch):
    def row(rr, carry):
        for c in range(HID // 16):
            sl = pl.ds(c * 16, 16)
            rows_o[rr, sl] = rows_d[rr, sl] + rows_s[rr, sl]
        return carry

    lax.fori_loop(0, ch, row, 0)


def _sc_gather_sum(tab_d, tab_s, idx_d, idx_s):
    return _sc_gather_combine(tab_d, tab_s, idx_d, idx_s, HID, _combine_sum)


def _make_geo_combine(ch):
    del ch

    def geo_combine(rows_d, rows_s, rows_o, ch):
        lane = lax.iota(jnp.int32, 16)

        def row(rr, carry):
            sl = pl.ds(0, 16)
            d = rows_d[rr, sl] - rows_s[rr, sl]
            rows_o[rr, sl] = jnp.where(lane == 9, rows_s[rr, sl], d)
            return carry

        lax.fori_loop(0, ch, row, 0)

    return geo_combine


def _sc_gather_geo(tab, idx_d, idx_s):
    return _sc_gather_combine(tab, tab, idx_d, idx_s, 16, _make_geo_combine(64))


def _sc_scatter_add(vals, idx, zeros_hbm):
    W = vals.shape[1]
    dt = vals.dtype
    ch = 64
    nch = PER_W // ch
    NB = 4
    rows_per_sub = N_PAD // NS

    @functools.partial(
        pl.kernel,
        out_type=jax.ShapeDtypeStruct((NC, N_PAD, W), dt),
        mesh=_sc_mesh(),
        scratch_types=[
            [pltpu.VMEM((ch,), jnp.int32)] * NB,
            [pltpu.VMEM((ch, W), dt)] * NB,
            pltpu.VMEM_SHARED((N_PAD, W), dt),
            [pltpu.SemaphoreType.DMA] * NB,
            [pltpu.SemaphoreType.DMA] * NB,
        ],
    )
    def k(v_hbm, i_hbm, z_hbm, out_hbm, iv, rows, acc, si, ss):
        cid = lax.axis_index("c")
        sid = lax.axis_index("s")
        wid = sid * NC + cid
        w0 = wid * PER_W
        sl = pl.ds(sid * rows_per_sub, rows_per_sub)

        def issue_loads(i, b):
            pltpu.async_copy(i_hbm.at[pl.ds(w0 + i * ch, ch)], iv[b], si[b])
            pltpu.async_copy(v_hbm.at[pl.ds(w0 + i * ch, ch)], rows[b], si[b])

        def drain_loads(b):
            pltpu.make_async_copy(i_hbm.at[pl.ds(0, ch)], iv[b], si[b]).wait()
            pltpu.make_async_copy(v_hbm.at[pl.ds(0, ch)], rows[b], si[b]).wait()

        def drain_scatter(b):
            pltpu.make_async_copy(rows[b], acc.at[iv[b]], ss[b]).wait()

        for q in range(NB):
            issue_loads(q, q)
        pltpu.sync_copy(z_hbm.at[sl], acc.at[sl])
        plsc.subcore_barrier()

        def body(s, carry):
            for q in range(NB):
                i = NB * s + q
                drain_loads(q)
                pltpu.async_copy(rows[q], acc.at[iv[q]], ss[q], add=True)
                j = i - 2
                bj = (q + 2) % NB

                @pl.when(j >= 0)
                def _():
                    drain_scatter(bj)

                    @pl.when(j + NB < nch)
                    def _():
                        issue_loads(j + NB, bj)
            return carry

        lax.fori_loop(0, nch // NB, body, 0)
        for jq in (nch - 2, nch - 1):
            drain_scatter(jq % NB)
        plsc.subcore_barrier()
        pltpu.sync_copy(acc.at[sl], out_hbm.at[cid, sl])

    return k(vals, idx, zeros_hbm)


def _sc_count(idx, zeros_hbm, ones_hbm):
    W = ones_hbm.shape[1]
    rows_per_sub = N_PAD // NS

    @functools.partial(
        pl.kernel,
        out_type=jax.ShapeDtypeStruct((NC, N_PAD, W), _f32),
        mesh=_sc_mesh(),
        scratch_types=[
            pltpu.VMEM((CH,), jnp.int32),
            pltpu.VMEM((CH, W), _f32),
            pltpu.VMEM_SHARED((N_PAD, W), _f32),
        ],
    )
    def k(i_hbm, z_hbm, o_hbm, out_hbm, iv, rows, acc):
        cid = lax.axis_index("c")
        sid = lax.axis_index("s")
        wid = sid * NC + cid
        sl = pl.ds(sid * rows_per_sub, rows_per_sub)
        pltpu.sync_copy(z_hbm.at[sl], acc.at[sl])
        pltpu.sync_copy(o_hbm, rows)
        plsc.subcore_barrier()

        def body(i, carry):
            base = wid * PER_W + i * CH
            pltpu.sync_copy(i_hbm.at[pl.ds(base, CH)], iv)
            pltpu.sync_copy(rows, acc.at[iv], add=True)
            return carry

        lax.fori_loop(0, N_CHUNK, body, 0)
        plsc.subcore_barrier()
        pltpu.sync_copy(acc.at[sl], out_hbm.at[cid, sl])

    return k(idx, zeros_hbm, ones_hbm)


def _full(shape):
    return pl.BlockSpec(shape, lambda i: tuple(0 for _ in shape))


def _embed_call(feats, w1, b1, w2, b2, wd, ws):
    BLK = 1024

    def body(f, w1r, b1r, w2r, b2r, wdr, wsr, h_o, pd_o, ps_o):
        x = f[...]
        h1 = jnp.maximum(
            jnp.dot(x, w1r[...], preferred_element_type=_f32) + b1r[...], 0.0)
        h = jnp.maximum(
            jnp.dot(h1, w2r[...], preferred_element_type=_f32) + b2r[...], 0.0)
        h_o[...] = h
        pd_o[...] = jnp.dot(h, wdr[...], preferred_element_type=_f32)
        ps_o[...] = jnp.dot(h, wsr[...], preferred_element_type=_f32)

    return pl.pallas_call(
        body,
        grid=(N_PAD // BLK,),
        in_specs=[
            pl.BlockSpec((BLK, 32), lambda i: (i, 0)),
            _full((32, HID)), _full((1, HID)),
            _full((HID, HID)), _full((1, HID)),
            _full((HID, HID)), _full((HID, HID)),
        ],
        out_specs=[pl.BlockSpec((BLK, HID), lambda i: (i, 0))] * 3,
        out_shape=[jax.ShapeDtypeStruct((N_PAD, HID), _f32)] * 3,
    )(feats, w1, b1, w2, b2, wd, ws)


def _geo_call(graw, dom_row):
    BLK = 2048

    def body(gr, domr, g_o):
        d = gr[...]
        dom = domr[...]
        w = d - dom * jnp.round(d / dom)
        ci = lax.broadcasted_iota(jnp.int32, d.shape, 1)
        g_o[...] = jnp.where(ci < 3, w, d)

    return pl.pallas_call(
        body,
        grid=(E_PAD // BLK,),
        in_specs=[
            pl.BlockSpec((BLK, 16), lambda i: (i, 0)),
            _full((1, 16)),
        ],
        out_specs=pl.BlockSpec((BLK, 16), lambda i: (i, 0)),
        out_shape=jax.ShapeDtypeStruct((E_PAD, 16), _f32),
    )(graw, dom_row)


def _edge_call(gsum, geo, wgeo, b1, w2, b2):
    BLK = 2048

    def body(gr, ger, wgr, b1r, w2r, b2r, m_o):
        h1 = jnp.maximum(
            gr[...]
            + jnp.dot(ger[...], wgr[...], preferred_element_type=_f32)
            + b1r[...], 0.0)
        m_o[...] = jnp.maximum(
            jnp.dot(h1, w2r[...], preferred_element_type=_f32) + b2r[...], 0.0)

    return pl.pallas_call(
        body,
        grid=(E_PAD // BLK,),
        in_specs=[
            pl.BlockSpec((BLK, HID), lambda i: (i, 0)),
            pl.BlockSpec((BLK, 16), lambda i: (i, 0)),
            _full((16, HID)), _full((1, HID)),
            _full((HID, HID)), _full((1, HID)),
        ],
        out_specs=pl.BlockSpec((BLK, HID), lambda i: (i, 0)),
        out_shape=jax.ShapeDtypeStruct((E_PAD, HID), _f32),
    )(gsum, geo, wgeo, b1, w2, b2)


def _node_call(h, aggp, degp, wa, wb, b1, w2, b2, g, be, wd, ws):
    BLK = 1024

    def body(hr, ar, dr, war, wbr, b1r, w2r, b2r, gr, ber, wdr, wsr,
             h_o, pd_o, ps_o):
        h = hr[...]
        deg = jnp.maximum(dr[0, :, 0:1] + dr[1, :, 0:1], 1.0)
        agg = (ar[0].astype(_f32) + ar[1].astype(_f32)) / deg
        u1 = jnp.maximum(
            jnp.dot(h, war[...], preferred_element_type=_f32)
            + jnp.dot(agg, wbr[...], preferred_element_type=_f32)
            + b1r[...], 0.0)
        u = jnp.dot(u1, w2r[...], preferred_element_type=_f32) + b2r[...]
        mu = jnp.mean(u, axis=-1, keepdims=True)
        var = jnp.mean((u - mu) ** 2, axis=-1, keepdims=True)
        un = (u - mu) * lax.rsqrt(var + 1e-5) * gr[...] + ber[...]
        hn = h + un
        h_o[...] = hn
        pd_o[...] = jnp.dot(hn, wdr[...], preferred_element_type=_f32)
        ps_o[...] = jnp.dot(hn, wsr[...], preferred_element_type=_f32)

    return pl.pallas_call(
        body,
        grid=(N_PAD // BLK,),
        in_specs=[
            pl.BlockSpec((BLK, HID), lambda i: (i, 0)),
            pl.BlockSpec((NC, BLK, HID), lambda i: (0, i, 0)),
            pl.BlockSpec((NC, BLK, HID), lambda i: (0, i, 0)),
            _full((HID, HID)), _full((HID, HID)), _full((1, HID)),
            _full((HID, HID)), _full((1, HID)),
            _full((1, HID)), _full((1, HID)),
            _full((HID, HID)), _full((HID, HID)),
        ],
        out_specs=[pl.BlockSpec((BLK, HID), lambda i: (i, 0))] * 3,
        out_shape=[jax.ShapeDtypeStruct((N_PAD, HID), _f32)] * 3,
    )(h, aggp, degp, wa, wb, b1, w2, b2, g, be, wd, ws)


def _head_call(h, nf, ow1, ob1, ow2p, ob2p, scale_row, dn_row):
    BLK = 1024

    def body(hr, nfr, w1r, b1r, w2r, b2r, scr, dnr, pred_o, hsum_o):
        i = pl.program_id(0)
        h = hr[...]
        o1 = jnp.maximum(
            jnp.dot(h, w1r[...], preferred_element_type=_f32) + b1r[...], 0.0)
        o = jnp.dot(o1, w2r[...], preferred_element_type=_f32) + b2r[...]
        base = o * scr[...] + nfr[...]
        dn = dnr[...]
        rem = base - jnp.floor(base / dn) * dn
        ci = lax.broadcasted_iota(jnp.int32, base.shape, 1)
        pred_o[...] = jnp.where(ci < 3, rem, base)
        ri = lax.broadcasted_iota(jnp.int32, (BLK, 1), 0) + i * BLK
        hm = jnp.where(ri < N, h, 0.0)
        part = jnp.sum(hm, axis=0, keepdims=True)

        @pl.when(i == 0)
        def _init():
            hsum_o[...] = jnp.zeros_like(hsum_o)

        hsum_o[...] += part

    return pl.pallas_call(
        body,
        grid=(N_PAD // BLK,),
        in_specs=[
            pl.BlockSpec((BLK, HID), lambda i: (i, 0)),
            pl.BlockSpec((BLK, HID), lambda i: (i, 0)),
            _full((HID, HID)), _full((1, HID)),
            _full((HID, HID)), _full((1, HID)),
            _full((1, HID)), _full((1, HID)),
        ],
        out_specs=[
            pl.BlockSpec((BLK, HID), lambda i: (i, 0)),
            pl.BlockSpec((1, HID), lambda i: (0, 0)),
        ],
        out_shape=[
            jax.ShapeDtypeStruct((N_PAD, HID), _f32),
            jax.ShapeDtypeStruct((1, HID), _f32),
        ],
    )(h, nf, ow1, ob1, ow2p, ob2p, scale_row, dn_row)


def _macro_call(hsum, w1, b1, w2p, b2p):
    def body(hs, w1r, b1r, w2r, b2r, o):
        hm = jnp.broadcast_to(hs[...] * (1.0 / N), (8, HID))
        z1 = jnp.maximum(
            jnp.dot(hm, w1r[...], preferred_element_type=_f32) + b1r[...], 0.0)
        o[...] = jnp.dot(z1, w2r[...], preferred_element_type=_f32) + b2r[...]

    return pl.pallas_call(
        body,
        out_shape=jax.ShapeDtypeStruct((8, HID), _f32),
    )(hsum, w1, b1, w2p, b2p)


def kernel(pos, v, r, t, x_global, domain, domain_next, t_next,
           edge_index, batch, params):
    gfeat = jnp.concatenate([domain, t, x_global, domain_next, t_next])

    src = edge_index[0]
    dst = edge_index[1]
    pad_e = E_PAD - E
    src_p = jnp.concatenate([src, jnp.zeros((pad_e,), jnp.int32)])
    dst_p = jnp.concatenate([dst, jnp.full((pad_e,), N, jnp.int32)])

    T = jnp.concatenate([pos, v, r], axis=1)
    T = jnp.pad(T, ((0, N_PAD - N), (0, HID - 10)))

    feats = jnp.concatenate([r, v, jnp.broadcast_to(gfeat, (N, 12))], axis=1)
    feats = jnp.pad(feats, ((0, N_PAD - N), (0, 32 - 19)))

    Wd_l, Ws_l, Wgeo_l, pb1_l, pW2_l, pb2_l = [], [], [], [], [], []
    A_l, B_l, sb1_l, sW2_l, sb2_l, g_l, be_l = [], [], [], [], [], [], []
    for lp in params["layers"]:
        W1 = lp["phi_W1"]
        Wd_l.append(W1[0:HID])
        Ws_l.append(W1[HID:2 * HID])
        Wgeo_l.append(jnp.pad(W1[2 * HID:2 * HID + 10], ((0, 6), (0, 0))))
        pb1_l.append((lp["phi_b1"] + gfeat @ W1[2 * HID + 10:])[None])
        pW2_l.append(lp["phi_W2"])
        pb2_l.append(lp["phi_b2"][None])
        A_l.append(lp["psi_W1"][0:HID])
        B_l.append(lp["psi_W1"][HID:2 * HID])
        sb1_l.append(lp["psi_b1"][None])
        sW2_l.append(lp["psi_W2"])
        sb2_l.append(lp["psi_b2"][None])
        g_l.append(lp["ln_g"][None])
        be_l.append(lp["ln_b"][None])

    emb_W1p = jnp.pad(params["emb_W1"], ((0, 32 - 19), (0, 0)))

    zeros_np32 = jnp.zeros((N_PAD, HID), _f32)
    ones_ch = jnp.ones((CH, HID), _f32)
    dom_row = jnp.concatenate([domain, jnp.ones((13,), _f32)])[None]

    geo_raw = _sc_gather_geo(T, dst_p, src_p)
    geo = _geo_call(geo_raw, dom_row)
    degp = _sc_count(dst_p, zeros_np32, ones_ch)
    h, Pd, Ps = _embed_call(feats, emb_W1p, params["emb_b1"][None],
                            params["emb_W2"], params["emb_b2"][None],
                            Wd_l[0], Ws_l[0])
    for l in range(6):
        Gsum = _sc_gather_sum(Pd, Ps, dst_p, src_p)
        m = _edge_call(Gsum, geo, Wgeo_l[l], pb1_l[l], pW2_l[l], pb2_l[l])
        aggp = _sc_scatter_add(m, dst_p, zeros_np32)
        nl = (l + 1) % 6
        h, Pd, Ps = _node_call(h, aggp, degp, A_l[l], B_l[l], sb1_l[l],
                               sW2_l[l], sb2_l[l], g_l[l], be_l[l],
                               Wd_l[nl], Ws_l[nl])

    nf = jnp.pad(jnp.concatenate([pos, v], axis=1),
                 ((0, N_PAD - N), (0, HID - 9)))
    scale_row = jnp.concatenate([
        jnp.full((6,), 0.001, _f32), jnp.full((3,), 0.01, _f32),
        jnp.zeros((HID - 9,), _f32)])[None]
    dn_row = jnp.concatenate([domain_next, jnp.ones((HID - 3,), _f32)])[None]
    ow2p = jnp.pad(params["out_W2"], ((0, 0), (0, HID - 9)))
    ob2p = jnp.pad(params["out_b2"], (0, HID - 9))[None]
    pred, hsum = _head_call(h, nf, params["out_W1"], params["out_b1"][None],
                            ow2p, ob2p, scale_row, dn_row)
    mw2p = jnp.pad(params["mac_W2"], ((0, 0), (0, HID - 3)))
    mb2p = jnp.pad(params["mac_b2"], (0, HID - 3))[None]
    macro8 = _macro_call(hsum, params["mac_W1"], params["mac_b1"][None],
                         mw2p, mb2p)

    return (pred[:N, 0:3], pred[:N, 3:9], macro8[0, 0:3])


# speedup vs baseline: 1.0918x; 1.0704x over previous
import functools

import jax
import jax.numpy as jnp
from jax import lax
from jax.experimental import pallas as pl
from jax.experimental.pallas import tpu as pltpu
from jax.experimental.pallas import tpu_sc as plsc

N = 10000
E = 160000
HID = 128
N_PAD = 10240
E_PAD = 163840
NC, NS = 2, 16
NW = NC * NS
CH = 128
PER_W = E_PAD // NW
N_CHUNK = PER_W // CH
_f32 = jnp.float32


def _sc_mesh():
    return plsc.VectorSubcoreMesh(core_axis_name="c", subcore_axis_name="s",
                                  num_cores=NC, num_subcores=NS)


def _sc_gather_combine(tab_d, tab_s, idx_d, idx_s, out_w, combine):
    W = tab_d.shape[1]
    dt = tab_d.dtype
    ch = 64
    E_loc = idx_d.shape[0]
    per_w = E_loc // NW
    nch = per_w // ch
    NB = 4
    out_t = jax.ShapeDtypeStruct((E_loc, out_w), dt)

    @functools.partial(
        pl.kernel,
        out_type=out_t,
        mesh=_sc_mesh(),
        scratch_types=[
            [pltpu.VMEM((ch,), jnp.int32)] * NB,
            [pltpu.VMEM((ch,), jnp.int32)] * NB,
            [pltpu.VMEM((ch, W), dt)] * NB,
            [pltpu.VMEM((ch, W), dt)] * NB,
            [pltpu.VMEM((ch, out_w), dt)] * NB,
            [pltpu.SemaphoreType.DMA] * NB,
            [pltpu.SemaphoreType.DMA] * NB,
            [pltpu.SemaphoreType.DMA] * NB,
        ],
    )
    def k(td_hbm, ts_hbm, id_hbm, is_hbm, go_hbm,
          iv_d, iv_s, rows_d, rows_s, rows_o, si, sg, so):
        wid = lax.axis_index("s") * NC + lax.axis_index("c")
        w0 = wid * per_w

        def issue_idx(i, b):
            pltpu.async_copy(id_hbm.at[pl.ds(w0 + i * ch, ch)], iv_d[b], si[b])
            pltpu.async_copy(is_hbm.at[pl.ds(w0 + i * ch, ch)], iv_s[b], si[b])

        def drain_idx(b):
            pltpu.make_async_copy(id_hbm.at[pl.ds(0, ch)], iv_d[b], si[b]).wait()
            pltpu.make_async_copy(is_hbm.at[pl.ds(0, ch)], iv_s[b], si[b]).wait()

        def issue_gather(b):
            pltpu.async_copy(td_hbm.at[iv_d[b]], rows_d[b], sg[b])
            pltpu.async_copy(ts_hbm.at[iv_s[b]], rows_s[b], sg[b])

        def drain_gather(b):
            pltpu.make_async_copy(td_hbm.at[iv_d[b]], rows_d[b], sg[b]).wait()
            pltpu.make_async_copy(ts_hbm.at[iv_s[b]], rows_s[b], sg[b]).wait()

        def issue_wb(j, b):
            pltpu.async_copy(rows_o[b], go_hbm.at[pl.ds(w0 + j * ch, ch)], so[b])

        def drain_wb(b):
            pltpu.make_async_copy(rows_o[b], go_hbm.at[pl.ds(0, ch)], so[b]).wait()

        for q in range(NB):
            issue_idx(q, q)

        def body(s, carry):
            for q in range(NB):
                i = NB * s + q

                @pl.when(s >= 1)
                def _():
                    drain_wb(q)

                drain_idx(q)
                issue_gather(q)
                j = i - 2
                bj = (q + 2) % NB

                @pl.when(j >= 0)
                def _():
                    drain_gather(bj)
                    combine(rows_d[bj], rows_s[bj], rows_o[bj], ch)
                    issue_wb(j, bj)

                    @pl.when(j + NB < nch)
                    def _():
                        issue_idx(j + NB, bj)
            return carry

        lax.fori_loop(0, nch // NB, body, 0)
        for jq in (nch - 2, nch - 1):
            b = jq % NB
            drain_gather(b)
            combine(rows_d[b], rows_s[b], rows_o[b], ch)
            issue_wb(jq, b)
        for b in range(NB):
            drain_wb(b)

    return k(tab_d, tab_s, idx_d, idx_s)


def _combine_sum(rows_d, rows_s, rows_o, ch):
    def row(rr, carry):
        for c in range(HID // 16):
            sl = pl.ds(c * 16, 16)
            rows_o[rr, sl] = rows_d[rr, sl] + rows_s[rr, sl]
        return carry

    lax.fori_loop(0, ch, row, 0)


def _sc_gather_sum(tab_d, tab_s, idx_d, idx_s):
    return _sc_gather_combine(tab_d, tab_s, idx_d, idx_s, HID, _combine_sum)


def _make_geo_combine(ch):
    del ch

    def geo_combine(rows_d, rows_s, rows_o, ch):
        lane = lax.iota(jnp.int32, 16)

        def row(rr, carry):
            sl = pl.ds(0, 16)
            d = rows_d[rr, sl] - rows_s[rr, sl]
            rows_o[rr, sl] = jnp.where(lane == 9, rows_s[rr, sl], d)
            return carry

        lax.fori_loop(0, ch, row, 0)

    return geo_combine


def _sc_gather_geo(tab, idx_d, idx_s):
    return _sc_gather_combine(tab, tab, idx_d, idx_s, 16, _make_geo_combine(64))


def _sc_scatter_add(vals, idx, zeros_hbm):
    W = vals.shape[1]
    dt = vals.dtype
    ch = 64
    E_loc = idx.shape[0]
    per_w = E_loc // NW
    nch = per_w // ch
    NB = 4
    rows_per_sub = N_PAD // NS

    @functools.partial(
        pl.kernel,
        out_type=jax.ShapeDtypeStruct((NC, N_PAD, W), dt),
        mesh=_sc_mesh(),
        scratch_types=[
            [pltpu.VMEM((ch,), jnp.int32)] * NB,
            [pltpu.VMEM((ch, W), dt)] * NB,
            pltpu.VMEM_SHARED((N_PAD, W), dt),
            [pltpu.SemaphoreType.DMA] * NB,
            [pltpu.SemaphoreType.DMA] * NB,
        ],
    )
    def k(v_hbm, i_hbm, z_hbm, out_hbm, iv, rows, acc, si, ss):
        cid = lax.axis_index("c")
        sid = lax.axis_index("s")
        wid = sid * NC + cid
        w0 = wid * per_w
        sl = pl.ds(sid * rows_per_sub, rows_per_sub)

        def issue_loads(i, b):
            pltpu.async_copy(i_hbm.at[pl.ds(w0 + i * ch, ch)], iv[b], si[b])
            pltpu.async_copy(v_hbm.at[pl.ds(w0 + i * ch, ch)], rows[b], si[b])

        def drain_loads(b):
            pltpu.make_async_copy(i_hbm.at[pl.ds(0, ch)], iv[b], si[b]).wait()
            pltpu.make_async_copy(v_hbm.at[pl.ds(0, ch)], rows[b], si[b]).wait()

        def drain_scatter(b):
            pltpu.make_async_copy(rows[b], acc.at[iv[b]], ss[b]).wait()

        for q in range(NB):
            issue_loads(q, q)
        pltpu.sync_copy(z_hbm.at[sl], acc.at[sl])
        plsc.subcore_barrier()

        def body(s, carry):
            for q in range(NB):
                i = NB * s + q
                drain_loads(q)
                pltpu.async_copy(rows[q], acc.at[iv[q]], ss[q], add=True)
                j = i - 2
                bj = (q + 2) % NB

                @pl.when(j >= 0)
                def _():
                    drain_scatter(bj)

                    @pl.when(j + NB < nch)
                    def _():
                        issue_loads(j + NB, bj)
            return carry

        lax.fori_loop(0, nch // NB, body, 0)
        for jq in (nch - 2, nch - 1):
            drain_scatter(jq % NB)
        plsc.subcore_barrier()
        pltpu.sync_copy(acc.at[sl], out_hbm.at[cid, sl])

    return k(vals, idx, zeros_hbm)


def _sc_count(idx, zeros_hbm, ones_hbm):
    W = ones_hbm.shape[1]
    rows_per_sub = N_PAD // NS

    @functools.partial(
        pl.kernel,
        out_type=jax.ShapeDtypeStruct((NC, N_PAD, W), _f32),
        mesh=_sc_mesh(),
        scratch_types=[
            pltpu.VMEM((CH,), jnp.int32),
            pltpu.VMEM((CH, W), _f32),
            pltpu.VMEM_SHARED((N_PAD, W), _f32),
        ],
    )
    def k(i_hbm, z_hbm, o_hbm, out_hbm, iv, rows, acc):
        cid = lax.axis_index("c")
        sid = lax.axis_index("s")
        wid = sid * NC + cid
        sl = pl.ds(sid * rows_per_sub, rows_per_sub)
        pltpu.sync_copy(z_hbm.at[sl], acc.at[sl])
        pltpu.sync_copy(o_hbm, rows)
        plsc.subcore_barrier()

        def body(i, carry):
            base = wid * PER_W + i * CH
            pltpu.sync_copy(i_hbm.at[pl.ds(base, CH)], iv)
            pltpu.sync_copy(rows, acc.at[iv], add=True)
            return carry

        lax.fori_loop(0, N_CHUNK, body, 0)
        plsc.subcore_barrier()
        pltpu.sync_copy(acc.at[sl], out_hbm.at[cid, sl])

    return k(idx, zeros_hbm, ones_hbm)


def _full(shape):
    return pl.BlockSpec(shape, lambda i: tuple(0 for _ in shape))


def _embed_call(feats, w1, b1, w2, b2, wd, ws):
    BLK = 1024

    def body(f, w1r, b1r, w2r, b2r, wdr, wsr, h_o, pd_o, ps_o):
        x = f[...]
        h1 = jnp.maximum(
            jnp.dot(x, w1r[...], preferred_element_type=_f32) + b1r[...], 0.0)
        h = jnp.maximum(
            jnp.dot(h1, w2r[...], preferred_element_type=_f32) + b2r[...], 0.0)
        h_o[...] = h
        pd_o[...] = jnp.dot(h, wdr[...], preferred_element_type=_f32)
        ps_o[...] = jnp.dot(h, wsr[...], preferred_element_type=_f32)

    return pl.pallas_call(
        body,
        grid=(N_PAD // BLK,),
        in_specs=[
            pl.BlockSpec((BLK, 32), lambda i: (i, 0)),
            _full((32, HID)), _full((1, HID)),
            _full((HID, HID)), _full((1, HID)),
            _full((HID, HID)), _full((HID, HID)),
        ],
        out_specs=[pl.BlockSpec((BLK, HID), lambda i: (i, 0))] * 3,
        out_shape=[jax.ShapeDtypeStruct((N_PAD, HID), _f32)] * 3,
    )(feats, w1, b1, w2, b2, wd, ws)


def _geo_call(graw, dom_row):
    BLK = 2048

    def body(gr, domr, g_o):
        d = gr[...]
        dom = domr[...]
        w = d - dom * jnp.round(d / dom)
        ci = lax.broadcasted_iota(jnp.int32, d.shape, 1)
        g_o[...] = jnp.where(ci < 3, w, d)

    return pl.pallas_call(
        body,
        grid=(E_PAD // BLK,),
        in_specs=[
            pl.BlockSpec((BLK, 16), lambda i: (i, 0)),
            _full((1, 16)),
        ],
        out_specs=pl.BlockSpec((BLK, 16), lambda i: (i, 0)),
        out_shape=jax.ShapeDtypeStruct((E_PAD, 16), _f32),
    )(graw, dom_row)


def _edge_call(gsum, geo, wgeo, b1, w2, b2):
    BLK = 2048
    E_loc = gsum.shape[0]

    def body(gr, ger, wgr, b1r, w2r, b2r, m_o):
        h1 = jnp.maximum(
            gr[...]
            + jnp.dot(ger[...], wgr[...], preferred_element_type=_f32)
            + b1r[...], 0.0)
        m_o[...] = jnp.maximum(
            jnp.dot(h1, w2r[...], preferred_element_type=_f32) + b2r[...], 0.0)

    return pl.pallas_call(
        body,
        grid=(E_loc // BLK,),
        in_specs=[
            pl.BlockSpec((BLK, HID), lambda i: (i, 0)),
            pl.BlockSpec((BLK, 16), lambda i: (i, 0)),
            _full((16, HID)), _full((1, HID)),
            _full((HID, HID)), _full((1, HID)),
        ],
        out_specs=pl.BlockSpec((BLK, HID), lambda i: (i, 0)),
        out_shape=jax.ShapeDtypeStruct((E_loc, HID), _f32),
    )(gsum, geo, wgeo, b1, w2, b2)


def _node_call(h, aggp, aggp2, degp, wa, wb, b1, w2, b2, g, be, wd, ws):
    BLK = 1024

    def body(hr, ar, ar2, dr, war, wbr, b1r, w2r, b2r, gr, ber, wdr, wsr,
             h_o, pd_o, ps_o):
        h = hr[...]
        deg = jnp.maximum(dr[0, :, 0:1] + dr[1, :, 0:1], 1.0)
        agg = (ar[0] + ar[1] + ar2[0] + ar2[1]) / deg
        u1 = jnp.maximum(
            jnp.dot(h, war[...], preferred_element_type=_f32)
            + jnp.dot(agg, wbr[...], preferred_element_type=_f32)
            + b1r[...], 0.0)
        u = jnp.dot(u1, w2r[...], preferred_element_type=_f32) + b2r[...]
        mu = jnp.mean(u, axis=-1, keepdims=True)
        var = jnp.mean((u - mu) ** 2, axis=-1, keepdims=True)
        un = (u - mu) * lax.rsqrt(var + 1e-5) * gr[...] + ber[...]
        hn = h + un
        h_o[...] = hn
        pd_o[...] = jnp.dot(hn, wdr[...], preferred_element_type=_f32)
        ps_o[...] = jnp.dot(hn, wsr[...], preferred_element_type=_f32)

    return pl.pallas_call(
        body,
        grid=(N_PAD // BLK,),
        in_specs=[
            pl.BlockSpec((BLK, HID), lambda i: (i, 0)),
            pl.BlockSpec((NC, BLK, HID), lambda i: (0, i, 0)),
            pl.BlockSpec((NC, BLK, HID), lambda i: (0, i, 0)),
            pl.BlockSpec((NC, BLK, HID), lambda i: (0, i, 0)),
            _full((HID, HID)), _full((HID, HID)), _full((1, HID)),
            _full((HID, HID)), _full((1, HID)),
            _full((1, HID)), _full((1, HID)),
            _full((HID, HID)), _full((HID, HID)),
        ],
        out_specs=[pl.BlockSpec((BLK, HID), lambda i: (i, 0))] * 3,
        out_shape=[jax.ShapeDtypeStruct((N_PAD, HID), _f32)] * 3,
    )(h, aggp, aggp2, degp, wa, wb, b1, w2, b2, g, be, wd, ws)


def _head_call(h, nf, ow1, ob1, ow2p, ob2p, scale_row, dn_row):
    BLK = 1024

    def body(hr, nfr, w1r, b1r, w2r, b2r, scr, dnr, pred_o, hsum_o):
        i = pl.program_id(0)
        h = hr[...]
        o1 = jnp.maximum(
            jnp.dot(h, w1r[...], preferred_element_type=_f32) + b1r[...], 0.0)
        o = jnp.dot(o1, w2r[...], preferred_element_type=_f32) + b2r[...]
        base = o * scr[...] + nfr[...]
        dn = dnr[...]
        rem = base - jnp.floor(base / dn) * dn
        ci = lax.broadcasted_iota(jnp.int32, base.shape, 1)
        pred_o[...] = jnp.where(ci < 3, rem, base)
        ri = lax.broadcasted_iota(jnp.int32, (BLK, 1), 0) + i * BLK
        hm = jnp.where(ri < N, h, 0.0)
        part = jnp.sum(hm, axis=0, keepdims=True)

        @pl.when(i == 0)
        def _init():
            hsum_o[...] = jnp.zeros_like(hsum_o)

        hsum_o[...] += part

    return pl.pallas_call(
        body,
        grid=(N_PAD // BLK,),
        in_specs=[
            pl.BlockSpec((BLK, HID), lambda i: (i, 0)),
            pl.BlockSpec((BLK, HID), lambda i: (i, 0)),
            _full((HID, HID)), _full((1, HID)),
            _full((HID, HID)), _full((1, HID)),
            _full((1, HID)), _full((1, HID)),
        ],
        out_specs=[
            pl.BlockSpec((BLK, HID), lambda i: (i, 0)),
            pl.BlockSpec((1, HID), lambda i: (0, 0)),
        ],
        out_shape=[
            jax.ShapeDtypeStruct((N_PAD, HID), _f32),
            jax.ShapeDtypeStruct((1, HID), _f32),
        ],
    )(h, nf, ow1, ob1, ow2p, ob2p, scale_row, dn_row)


def _macro_call(hsum, w1, b1, w2p, b2p):
    def body(hs, w1r, b1r, w2r, b2r, o):
        hm = jnp.broadcast_to(hs[...] * (1.0 / N), (8, HID))
        z1 = jnp.maximum(
            jnp.dot(hm, w1r[...], preferred_element_type=_f32) + b1r[...], 0.0)
        o[...] = jnp.dot(z1, w2r[...], preferred_element_type=_f32) + b2r[...]

    return pl.pallas_call(
        body,
        out_shape=jax.ShapeDtypeStruct((8, HID), _f32),
    )(hsum, w1, b1, w2p, b2p)


def kernel(pos, v, r, t, x_global, domain, domain_next, t_next,
           edge_index, batch, params):
    gfeat = jnp.concatenate([domain, t, x_global, domain_next, t_next])

    src = edge_index[0]
    dst = edge_index[1]
    pad_e = E_PAD - E
    src_p = jnp.concatenate([src, jnp.zeros((pad_e,), jnp.int32)])
    dst_p = jnp.concatenate([dst, jnp.full((pad_e,), N, jnp.int32)])

    T = jnp.concatenate([pos, v, r], axis=1)
    T = jnp.pad(T, ((0, N_PAD - N), (0, HID - 10)))

    feats = jnp.concatenate([r, v, jnp.broadcast_to(gfeat, (N, 12))], axis=1)
    feats = jnp.pad(feats, ((0, N_PAD - N), (0, 32 - 19)))

    Wd_l, Ws_l, Wgeo_l, pb1_l, pW2_l, pb2_l = [], [], [], [], [], []
    A_l, B_l, sb1_l, sW2_l, sb2_l, g_l, be_l = [], [], [], [], [], [], []
    for lp in params["layers"]:
        W1 = lp["phi_W1"]
        Wd_l.append(W1[0:HID])
        Ws_l.append(W1[HID:2 * HID])
        Wgeo_l.append(jnp.pad(W1[2 * HID:2 * HID + 10], ((0, 6), (0, 0))))
        pb1_l.append((lp["phi_b1"] + gfeat @ W1[2 * HID + 10:])[None])
        pW2_l.append(lp["phi_W2"])
        pb2_l.append(lp["phi_b2"][None])
        A_l.append(lp["psi_W1"][0:HID])
        B_l.append(lp["psi_W1"][HID:2 * HID])
        sb1_l.append(lp["psi_b1"][None])
        sW2_l.append(lp["psi_W2"])
        sb2_l.append(lp["psi_b2"][None])
        g_l.append(lp["ln_g"][None])
        be_l.append(lp["ln_b"][None])

    emb_W1p = jnp.pad(params["emb_W1"], ((0, 32 - 19), (0, 0)))

    zeros_np32 = jnp.zeros((N_PAD, HID), _f32)
    ones_ch = jnp.ones((CH, HID), _f32)
    dom_row = jnp.concatenate([domain, jnp.ones((13,), _f32)])[None]

    geo_raw = _sc_gather_geo(T, dst_p, src_p)
    geo = _geo_call(geo_raw, dom_row)
    degp = _sc_count(dst_p, zeros_np32, ones_ch)
    h, Pd, Ps = _embed_call(feats, emb_W1p, params["emb_b1"][None],
                            params["emb_W2"], params["emb_b2"][None],
                            Wd_l[0], Ws_l[0])
    EH = E_PAD // 2
    dst_a, dst_b = dst_p[:EH], dst_p[EH:]
    src_a, src_b = src_p[:EH], src_p[EH:]
    geo_a, geo_b = geo[:EH], geo[EH:]

    gs_a = _sc_gather_sum(Pd, Ps, dst_a, src_a)
    for l in range(6):
        m_a = _edge_call(gs_a, geo_a, Wgeo_l[l], pb1_l[l], pW2_l[l], pb2_l[l])
        gs_b = _sc_gather_sum(Pd, Ps, dst_b, src_b)
        agg_a = _sc_scatter_add(m_a, dst_a, zeros_np32)
        m_b = _edge_call(gs_b, geo_b, Wgeo_l[l], pb1_l[l], pW2_l[l], pb2_l[l])
        agg_b = _sc_scatter_add(m_b, dst_b, zeros_np32)
        nl = (l + 1) % 6
        h, Pd, Ps = _node_call(h, agg_a, agg_b, degp, A_l[l], B_l[l],
                               sb1_l[l], sW2_l[l], sb2_l[l], g_l[l], be_l[l],
                               Wd_l[nl], Ws_l[nl])
        if l < 5:
            gs_a = _sc_gather_sum(Pd, Ps, dst_a, src_a)

    nf = jnp.pad(jnp.concatenate([pos, v], axis=1),
                 ((0, N_PAD - N), (0, HID - 9)))
    scale_row = jnp.concatenate([
        jnp.full((6,), 0.001, _f32), jnp.full((3,), 0.01, _f32),
        jnp.zeros((HID - 9,), _f32)])[None]
    dn_row = jnp.concatenate([domain_next, jnp.ones((HID - 3,), _f32)])[None]
    ow2p = jnp.pad(params["out_W2"], ((0, 0), (0, HID - 9)))
    ob2p = jnp.pad(params["out_b2"], (0, HID - 9))[None]
    pred, hsum = _head_call(h, nf, params["out_W1"], params["out_b1"][None],
                            ow2p, ob2p, scale_row, dn_row)
    mw2p = jnp.pad(params["mac_W2"], ((0, 0), (0, HID - 3)))
    mb2p = jnp.pad(params["mac_b2"], (0, HID - 3))[None]
    macro8 = _macro_call(hsum, params["mac_W1"], params["mac_b1"][None],
                         mw2p, mb2p)

    return (pred[:N, 0:3], pred[:N, 3:9], macro8[0, 0:3])
